# baseline (speedup 1.0000x reference)
import jax
import jax.numpy as jnp
from jax import lax
from jax.experimental import pallas as pl
from jax.experimental.pallas import tpu as pltpu
from jax.experimental.pallas import tpu_sc as plsc

_VOCAB = 100000
_D = 64
_B = 16384
_H = 50

_NC = 2
_NS = 16
_NW = _NC * _NS
_BPW = _B // _NW
_L = 16
_G = _BPW // _L
_Q = 4
_QL = _BPW // _Q


def _bag_body(ids_hbm, table_hbm, out_hbm, idx_nat, idx_t, acc_v, sem):
    wid = lax.axis_index("s") * _NC + lax.axis_index("c")
    base = wid * _BPW

    idx_stage = pltpu.make_async_copy(
        ids_hbm.at[pl.ds(base, _BPW)], idx_nat, sem)
    idx_stage.start()

    zeros = jnp.zeros((_L,), jnp.float32)

    def _zero(b, carry):
        for k in range(_D // _L):
            acc_v[b, pl.ds(k * _L, _L)] = zeros
        return carry

    lax.fori_loop(0, _BPW, _zero, 0)
    idx_stage.wait()

    lanes = lax.iota(jnp.int32, _L)

    def _step(j, carry):
        cols = jnp.full((_L,), j, jnp.int32)
        for g in range(_G):
            v = plsc.load_gather(idx_nat, [lanes + g * _L, cols])
            idx_t[j, pl.ds(g * _L, _L)] = v
        for q in range(_Q):
            pltpu.async_copy(
                table_hbm.at[idx_t.at[j, pl.ds(q * _QL, _QL)]],
                acc_v.at[pl.ds(q * _QL, _QL)],
                sem,
                add=True,
            )
        return carry

    lax.fori_loop(0, _H, _step, 0)

    def _drain(j, carry):
        pltpu.make_async_copy(table_hbm.at[idx_t.at[0]], acc_v, sem).wait()
        return carry

    lax.fori_loop(0, _H, _drain, 0)

    pltpu.sync_copy(acc_v, out_hbm.at[pl.ds(base, _BPW)])


_bag = pl.kernel(
    _bag_body,
    out_type=jax.ShapeDtypeStruct((_B, _D), jnp.float32),
    mesh=plsc.VectorSubcoreMesh(core_axis_name="c", subcore_axis_name="s"),
    scratch_types=[
        pltpu.VMEM((_BPW, _H), jnp.int32),
        pltpu.VMEM((_H, _BPW), jnp.int32),
        pltpu.VMEM((_BPW, _D), jnp.float32),
        pltpu.SemaphoreType.DMA,
    ],
    compiler_params=pltpu.CompilerParams(
        use_tc_tiling_on_sc=False, needs_layout_passes=False),
)


def kernel(ingredient_ids, embedding_table):
    return _bag(ingredient_ids.astype(jnp.int32), embedding_table)


# speedup vs baseline: 1.1063x; 1.1063x over previous
import jax
import jax.numpy as jnp
from jax import lax
from jax.experimental import pallas as pl
from jax.experimental.pallas import tpu as pltpu
from jax.experimental.pallas import tpu_sc as plsc

_VOCAB = 100000
_D = 64
_B = 16384
_H = 50

_NC = 2
_NS = 16
_NW = _NC * _NS
_BPW = _B // _NW
_Q = 1
_QL = _BPW // _Q


def _bag_body(ids_t_hbm, table_hbm, out_hbm, idx_v, acc_v, sem):
    wid = lax.axis_index("s") * _NC + lax.axis_index("c")
    base = wid * _BPW

    pltpu.sync_copy(ids_t_hbm.at[:, pl.ds(base, _BPW)], idx_v)

    for q in range(_Q):
        pltpu.async_copy(
            table_hbm.at[idx_v.at[0, pl.ds(q * _QL, _QL)]],
            acc_v.at[pl.ds(q * _QL, _QL)],
            sem,
        )
    pltpu.make_async_copy(table_hbm.at[idx_v.at[0]], acc_v, sem).wait()

    def _fire(j, carry):
        for q in range(_Q):
            pltpu.async_copy(
                table_hbm.at[idx_v.at[j, pl.ds(q * _QL, _QL)]],
                acc_v.at[pl.ds(q * _QL, _QL)],
                sem,
                add=True,
            )
        return carry

    lax.fori_loop(1, _H, _fire, 0)

    def _drain(j, carry):
        pltpu.make_async_copy(table_hbm.at[idx_v.at[0]], acc_v, sem).wait()
        return carry

    lax.fori_loop(1, _H, _drain, 0)

    pltpu.sync_copy(acc_v, out_hbm.at[pl.ds(base, _BPW)])


_bag = pl.kernel(
    _bag_body,
    out_type=jax.ShapeDtypeStruct((_B, _D), jnp.float32),
    mesh=plsc.VectorSubcoreMesh(core_axis_name="c", subcore_axis_name="s"),
    scratch_types=[
        pltpu.VMEM((_H, _BPW), jnp.int32),
        pltpu.VMEM((_BPW, _D), jnp.float32),
        pltpu.SemaphoreType.DMA,
    ],
    compiler_params=pltpu.CompilerParams(use_tc_tiling_on_sc=False),
)


def kernel(ingredient_ids, embedding_table):
    ids_t = jnp.transpose(ingredient_ids.astype(jnp.int32))
    t2 = lax.optimization_barrier(jnp.reshape(embedding_table, (_VOCAB // 2, 2 * _D)))
    table_lin = jnp.reshape(t2, (_VOCAB, _D))
    return _bag(ids_t, table_lin)
